# baseline (speedup 1.0000x reference)
import functools

import jax
import jax.numpy as jnp
from jax import lax
from jax.experimental import pallas as pl
from jax.experimental.pallas import tpu as pltpu
from jax.experimental.pallas import tpu_sc as plsc

NC = 2
NS = 16
NW = NC * NS

NROW = 16384
RL = 50
D = 32
IB = 128
NBPW = NROW // (IB * NW)
L = 16
DEPTH = 5
NG = RL // DEPTH

_mesh = plsc.VectorSubcoreMesh(core_axis_name="c", subcore_axis_name="s")


@functools.partial(
    pl.kernel,
    mesh=_mesh,
    compiler_params=pltpu.CompilerParams(use_tc_tiling_on_sc=False,
                                         needs_layout_passes=False),
    out_type=jax.ShapeDtypeStruct((RL, D // 8, NROW // IB, 8 * IB), jnp.float32),
    scratch_types=(
        [pltpu.VMEM((IB, RL), jnp.int32),
         pltpu.VMEM((RL, IB), jnp.int32)]
        + [pltpu.VMEM((IB, D), jnp.float32)] * DEPTH
        + [pltpu.VMEM((D * IB,), jnp.float32)] * DEPTH
        + [pltpu.SemaphoreType.DMA] * (2 * DEPTH)
    ),
)
def _embed(idx_hbm, tbl_hbm, out_hbm, xv, idxt, *rest):
    gbufs = rest[:DEPTH]
    tbufs = rest[DEPTH:2 * DEPTH]
    gsems = rest[2 * DEPTH:3 * DEPTH]
    osems = rest[3 * DEPTH:4 * DEPTH]

    wid = lax.axis_index("s") * NC + lax.axis_index("c")

    iota = lax.iota(jnp.int32, L)
    cvecs = [(c0 + iota) * IB for c0 in (0, L)]
    zero = iota - iota

    def transpose_rows(gb, tb):
        def trow(i1, carry):
            for h in range(2):
                vals = gb[i1, pl.ds(h * L, L)]
                plsc.store_scatter(tb, [cvecs[h] + i1], vals)
            return carry
        lax.fori_loop(0, IB, trow, 0, unroll=8)

    def issue_gather(j, p):
        pltpu.async_copy(tbl_hbm.at[idxt.at[j]], gbufs[p], gsems[p])

    def drain_gather(p):
        pltpu.make_async_copy(tbl_hbm.at[pl.ds(0, IB), :], gbufs[p],
                              gsems[p]).wait()

    def start_write(j, ibg, p):
        for cb in range(D // 8):
            pltpu.async_copy(tbufs[p].at[pl.ds(cb * 8 * IB, 8 * IB)],
                             out_hbm.at[j, cb, ibg], osems[p])

    def wait_write(p):
        for cb in range(D // 8):
            pltpu.make_async_copy(tbufs[p].at[pl.ds(cb * 8 * IB, 8 * IB)],
                                  out_hbm.at[0, cb, 0], osems[p]).wait()

    def block(b, carry):
        i0 = (wid * NBPW + b) * IB
        ibg = wid * NBPW + b
        pltpu.sync_copy(idx_hbm.at[pl.ds(i0, IB), :], xv)

        def tj(j, carry2):
            for ch in range(IB // L):
                vals = plsc.load_gather(xv, [iota + ch * L, zero + j])
                idxt[j, pl.ds(ch * L, L)] = vals
            return carry2
        lax.fori_loop(0, RL, tj, 0)

        for p in range(DEPTH):
            issue_gather(p, p)

        def group(g, carry2):
            for p in range(DEPTH):
                j = g * DEPTH + p
                drain_gather(p)

                @pl.when(g > 0)
                def _():
                    wait_write(p)

                transpose_rows(gbufs[p], tbufs[p])
                start_write(j, ibg, p)

                @pl.when(g < NG - 1)
                def _():
                    issue_gather(j + DEPTH, p)
            return carry2

        lax.fori_loop(0, NG, group, 0)
        for p in range(DEPTH):
            wait_write(p)
        return carry

    lax.fori_loop(0, NBPW, block, 0)


def kernel(x, weight):
    res = _embed(x.astype(jnp.int32), weight)
    res5 = res.reshape((RL, D // 8, NROW // IB, 8, IB))
    return res5.transpose((2, 4, 0, 1, 3)).reshape((NROW, RL, D))


# speedup vs baseline: 1.2978x; 1.2978x over previous
import functools

import jax
import jax.numpy as jnp
from jax import lax
from jax.experimental import pallas as pl
from jax.experimental.pallas import tpu as pltpu
from jax.experimental.pallas import tpu_sc as plsc

NC = 2
NS = 16
NW = NC * NS

NROW = 16384
RL = 50
D = 32
IB = 128
NBPW = NROW // (IB * NW)
L = 16
DEPTH = 5
NG = RL // DEPTH

_mesh = plsc.VectorSubcoreMesh(core_axis_name="c", subcore_axis_name="s")


@functools.partial(
    pl.kernel,
    mesh=_mesh,
    compiler_params=pltpu.CompilerParams(use_tc_tiling_on_sc=False,
                                         needs_layout_passes=False),
    out_type=jax.ShapeDtypeStruct((RL, D // 8, NROW // IB, 8, IB), jnp.float32),
    scratch_types=(
        [pltpu.VMEM((IB, RL), jnp.int32),
         pltpu.VMEM((RL, IB), jnp.int32)]
        + [pltpu.VMEM((IB, D), jnp.float32)] * DEPTH
        + [pltpu.VMEM((D, IB + 9), jnp.float32)] * DEPTH
        + [pltpu.SemaphoreType.DMA] * (2 * DEPTH)
    ),
)
def _embed(idx_hbm, tbl_hbm, out_hbm, xv, idxt, *rest):
    gbufs = rest[:DEPTH]
    tbufs = rest[DEPTH:2 * DEPTH]
    gsems = rest[2 * DEPTH:3 * DEPTH]
    osems = rest[3 * DEPTH:4 * DEPTH]

    wid = lax.axis_index("s") * NC + lax.axis_index("c")

    iota = lax.iota(jnp.int32, L)
    cvecs = [c0 + iota for c0 in (0, L)]
    zero = iota - iota

    def transpose_rows(gb, tb):
        def trow(i1, carry):
            ivec = zero + i1
            for h in range(2):
                vals = gb[i1, pl.ds(h * L, L)]
                plsc.store_scatter(tb, [cvecs[h], ivec], vals)
            return carry
        lax.fori_loop(0, IB, trow, 0, unroll=8)

    def issue_gather(j, p):
        pltpu.async_copy(tbl_hbm.at[idxt.at[j]], gbufs[p], gsems[p])

    def drain_gather(p):
        pltpu.make_async_copy(tbl_hbm.at[pl.ds(0, IB), :], gbufs[p],
                              gsems[p]).wait()

    def start_write(j, ibg, p):
        for cb in range(D // 8):
            pltpu.async_copy(tbufs[p].at[pl.ds(cb * 8, 8), pl.ds(0, IB)],
                             out_hbm.at[j, cb, ibg], osems[p])

    def wait_write(p):
        for cb in range(D // 8):
            pltpu.make_async_copy(tbufs[p].at[pl.ds(cb * 8, 8), pl.ds(0, IB)],
                                  out_hbm.at[0, cb, 0], osems[p]).wait()

    def block(b, carry):
        i0 = (wid * NBPW + b) * IB
        ibg = wid * NBPW + b
        pltpu.sync_copy(idx_hbm.at[pl.ds(i0, IB), :], xv)

        def tj(j, carry2):
            for ch in range(IB // L):
                vals = plsc.load_gather(xv, [iota + ch * L, zero + j])
                idxt[j, pl.ds(ch * L, L)] = vals
            return carry2
        lax.fori_loop(0, RL, tj, 0)

        for p in range(DEPTH):
            issue_gather(p, p)

        def group(g, carry2):
            for p in range(DEPTH):
                j = g * DEPTH + p
                drain_gather(p)

                @pl.when(g > 0)
                def _():
                    wait_write(p)

                transpose_rows(gbufs[p], tbufs[p])
                start_write(j, ibg, p)

                @pl.when(g < NG - 1)
                def _():
                    issue_gather(j + DEPTH, p)
            return carry2

        lax.fori_loop(0, NG, group, 0)
        for p in range(DEPTH):
            wait_write(p)
        return carry

    lax.fori_loop(0, NBPW, block, 0)


def kernel(x, weight):
    res5 = _embed(x.astype(jnp.int32), weight)
    return res5.transpose((2, 4, 0, 1, 3)).reshape((NROW, RL, D))


# speedup vs baseline: 1.2980x; 1.0001x over previous
import functools

import jax
import jax.numpy as jnp
from jax import lax
from jax.experimental import pallas as pl
from jax.experimental.pallas import tpu as pltpu
from jax.experimental.pallas import tpu_sc as plsc

NC = 2
NS = 16
NW = NC * NS

NROW = 16384
RL = 50
D = 32
IB = 128
NBPW = NROW // (IB * NW)
L = 16
DEPTH = 5
NG = RL // DEPTH

_mesh = plsc.VectorSubcoreMesh(core_axis_name="c", subcore_axis_name="s")


@functools.partial(
    pl.kernel,
    mesh=_mesh,
    compiler_params=pltpu.CompilerParams(use_tc_tiling_on_sc=False,
                                         needs_layout_passes=False),
    out_type=jax.ShapeDtypeStruct((RL, D // 8, NROW // IB, 8, IB), jnp.float32),
    scratch_types=(
        [pltpu.VMEM((IB, RL), jnp.int32),
         pltpu.VMEM((RL, IB), jnp.int32)]
        + [pltpu.VMEM((IB, 128), jnp.float32)] * DEPTH
        + [pltpu.VMEM((D, IB + 9), jnp.float32)] * DEPTH
        + [pltpu.SemaphoreType.DMA] * (2 * DEPTH)
    ),
)
def _embed(idx_hbm, tbl_hbm, out_hbm, xv, idxt, *rest):
    gbufs = rest[:DEPTH]
    tbufs = rest[DEPTH:2 * DEPTH]
    gsems = rest[2 * DEPTH:3 * DEPTH]
    osems = rest[3 * DEPTH:4 * DEPTH]

    wid = lax.axis_index("s") * NC + lax.axis_index("c")

    iota = lax.iota(jnp.int32, L)
    cvecs = [c0 + iota for c0 in (0, L)]
    zero = iota - iota

    def transpose_rows(gb, tb):
        def trow(i1, carry):
            ivec = zero + i1
            for h in range(2):
                vals = gb[i1, pl.ds(h * L, L)]
                plsc.store_scatter(tb, [cvecs[h], ivec], vals)
            return carry
        lax.fori_loop(0, IB, trow, 0, unroll=8)

    def issue_gather(j, p):
        pltpu.async_copy(tbl_hbm.at[idxt.at[j]], gbufs[p], gsems[p])

    def drain_gather(p):
        pltpu.make_async_copy(tbl_hbm.at[pl.ds(0, IB), :], gbufs[p],
                              gsems[p]).wait()

    def start_write(j, ibg, p):
        for cb in range(D // 8):
            pltpu.async_copy(tbufs[p].at[pl.ds(cb * 8, 8), pl.ds(0, IB)],
                             out_hbm.at[j, cb, ibg], osems[p])

    def wait_write(p):
        for cb in range(D // 8):
            pltpu.make_async_copy(tbufs[p].at[pl.ds(cb * 8, 8), pl.ds(0, IB)],
                                  out_hbm.at[0, cb, 0], osems[p]).wait()

    def block(b, carry):
        i0 = (wid * NBPW + b) * IB
        ibg = wid * NBPW + b
        pltpu.sync_copy(idx_hbm.at[pl.ds(i0, IB), :], xv)

        def tj(j, carry2):
            for ch in range(IB // L):
                vals = plsc.load_gather(xv, [iota + ch * L, zero + j])
                idxt[j, pl.ds(ch * L, L)] = vals
            return carry2
        lax.fori_loop(0, RL, tj, 0)

        for p in range(DEPTH):
            issue_gather(p, p)

        def group(g, carry2):
            for p in range(DEPTH):
                j = g * DEPTH + p
                drain_gather(p)

                @pl.when(g > 0)
                def _():
                    wait_write(p)

                transpose_rows(gbufs[p], tbufs[p])
                start_write(j, ibg, p)

                @pl.when(g < NG - 1)
                def _():
                    issue_gather(j + DEPTH, p)
            return carry2

        lax.fori_loop(0, NG, group, 0)
        for p in range(DEPTH):
            wait_write(p)
        return carry

    lax.fori_loop(0, NBPW, block, 0)


def kernel(x, weight):
    wpad = jnp.pad(weight, ((0, 0), (0, 128 - D)))
    res5 = _embed(x.astype(jnp.int32), wpad)
    return res5.transpose((2, 4, 0, 1, 3)).reshape((NROW, RL, D))
